# Initial kernel scaffold; baseline (speedup 1.0000x reference)
import jax
import jax.numpy as jnp
from jax.experimental import pallas as pl


def kernel(cosine, label):
    raise NotImplementedError("write your pallas kernel here")


# speedup vs baseline: 6.3063x; 6.3063x over previous
import math

import jax
import jax.numpy as jnp
from jax.experimental import pallas as pl

_S = 64.0
_COS_M = math.cos(0.5)
_SIN_M = math.sin(0.5)


def _arcface_body(lab_ref, cos_ref, out_ref):
    c = cos_ref[...]
    lab = lab_ref[...]
    bb, bc = c.shape
    cols = jax.lax.broadcasted_iota(jnp.int32, (bb, bc), 1)
    s2 = jnp.maximum(1.0 - c * c, 0.0)
    fix = _S * (c * _COS_M - jnp.sqrt(s2) * _SIN_M)
    out_ref[...] = jnp.where(cols == lab, fix, _S * c)


def kernel(cosine, label):
    B, C = cosine.shape
    bb = 8
    lab2 = label.reshape(B, 1)
    return pl.pallas_call(
        _arcface_body,
        grid=(B // bb,),
        in_specs=[
            pl.BlockSpec((bb, 1), lambda i: (i, 0)),
            pl.BlockSpec((bb, C), lambda i: (i, 0)),
        ],
        out_specs=pl.BlockSpec((bb, C), lambda i: (i, 0)),
        out_shape=jax.ShapeDtypeStruct((B, C), cosine.dtype),
    )(lab2, cosine)
